# baseline (speedup 1.0000x reference)
import functools

import jax
import jax.numpy as jnp
from jax import lax
from jax.experimental import pallas as pl
from jax.experimental.pallas import tpu as pltpu
from jax.experimental.pallas import tpu_sc as plsc

L = 16


def _iota():
    return lax.iota(jnp.int32, L)


def _splat(x):
    return jnp.zeros((L,), jnp.int32) + x


@functools.cache
def _make_detile(V, D):
    info = plsc.get_sparse_core_info()
    nw = info.num_cores * info.num_subcores
    nb_full = V // 128
    tail = V - nb_full * 128
    per_w, extra = divmod(nb_full, nw)
    mesh = plsc.VectorSubcoreMesh(core_axis_name="c", subcore_axis_name="s")

    @functools.partial(
        pl.kernel,
        mesh=mesh,
        out_type=jax.ShapeDtypeStruct((V // 2, 2 * D), jnp.float32),
        scratch_types=[
            pltpu.VMEM((D, 128), jnp.float32),
            pltpu.VMEM((D, 128), jnp.float32),
            pltpu.VMEM((64, 128), jnp.float32),
            pltpu.VMEM((64, 128), jnp.float32),
            pltpu.SemaphoreType.DMA,
            pltpu.SemaphoreType.DMA,
            pltpu.SemaphoreType.DMA,
            pltpu.SemaphoreType.DMA,
        ],
        compiler_params=pltpu.CompilerParams(
            use_tc_tiling_on_sc=True, needs_layout_passes=False
        ),
    )
    def k(tableT_hbm, tail_hbm, lt_hbm, inA, inB, rmA, rmB, siA, siB, soA, soB):
        wid = lax.axis_index("s") * info.num_cores + lax.axis_index("c")
        base = wid * per_w + jnp.minimum(wid, extra)
        cnt = per_w + (wid < extra).astype(jnp.int32)
        ridx = [_iota() + 16 * u for u in range(D // 16)]

        def start_in(t, buf, sem):
            j = base + jnp.minimum(t, cnt - 1)
            pltpu.async_copy(
                tableT_hbm.at[pl.ds(0, D), pl.ds(j * 128, 128)], buf, sem
            )

        def wait_in(buf, sem):
            pltpu.make_async_copy(
                tableT_hbm.at[pl.ds(0, D), pl.ds(0, 128)], buf, sem
            ).wait()

        def start_out(t, rm, sem):
            pltpu.async_copy(rm, lt_hbm.at[pl.ds((base + t) * 64, 64)], sem)

        def drain_out(rm, sem):
            pltpu.make_async_copy(rm, lt_hbm.at[pl.ds(0, 64)], sem).wait()

        def transpose(in_v, rm_v):
            @plsc.parallel_loop(0, 64, unroll=8)
            def body(kk):
                ce = _splat(2 * kk)
                co = ce + 1
                for u in range(2 * D // 16):
                    cv = ce if u < D // 16 else co
                    vals = plsc.load_gather(in_v, [ridx[u % (D // 16)], cv])
                    rm_v[kk, pl.ds(16 * u, 16)] = vals

        start_in(0, inA, siA)
        start_in(1, inB, siB)
        wait_in(inA, siA)
        transpose(inA, rmA)
        start_out(0, rmA, soA)
        start_in(2, inA, siA)
        wait_in(inB, siB)
        transpose(inB, rmB)
        start_out(1, rmB, soB)
        start_in(3, inB, siB)

        def pair(i, carry):
            t0 = 2 * i
            wait_in(inA, siA)
            drain_out(rmA, soA)
            transpose(inA, rmA)
            start_out(t0, rmA, soA)
            start_in(t0 + 2, inA, siA)
            wait_in(inB, siB)
            drain_out(rmB, soB)
            transpose(inB, rmB)
            start_out(t0 + 1, rmB, soB)
            start_in(t0 + 3, inB, siB)
            return carry

        lax.fori_loop(1, cnt // 2, pair, 0)
        wait_in(inA, siA)
        wait_in(inB, siB)

        @pl.when(cnt % 2 == 1)
        def _():
            drain_out(rmA, soA)
            transpose(inA, rmA)
            start_out(cnt - 1, rmA, soA)

        drain_out(rmA, soA)
        drain_out(rmB, soB)

        if tail:
            @pl.when(wid == nw - 1)
            def _():
                pltpu.async_copy(
                    tail_hbm, rmA.at[pl.ds(0, tail // 2)], siA
                ).wait()
                pltpu.async_copy(
                    rmA.at[pl.ds(0, tail // 2)],
                    lt_hbm.at[pl.ds(nb_full * 64, tail // 2)],
                    soA,
                ).wait()

    return k


@functools.cache
def _make_gather(BATCH, SEQ, D, V):
    info = plsc.get_sparse_core_info()
    nw = info.num_cores * info.num_subcores
    jblocks = BATCH // 128
    per_w = SEQ * jblocks // nw
    mesh = plsc.VectorSubcoreMesh(core_axis_name="c", subcore_axis_name="s")

    @functools.partial(
        pl.kernel,
        mesh=mesh,
        out_type=jax.ShapeDtypeStruct((SEQ, D, BATCH), jnp.float32),
        scratch_types=[
            pltpu.VMEM((128,), jnp.int32),
            pltpu.VMEM((128,), jnp.int32),
            pltpu.VMEM((128,), jnp.int32),
            pltpu.VMEM((128, 2 * D), jnp.float32),
            pltpu.VMEM((128, 2 * D), jnp.float32),
            pltpu.VMEM((D, 128), jnp.float32),
            pltpu.VMEM((D, 128), jnp.float32),
            pltpu.SemaphoreType.DMA,
            pltpu.SemaphoreType.DMA,
            pltpu.SemaphoreType.DMA,
            pltpu.SemaphoreType.DMA,
        ],
        compiler_params=pltpu.CompilerParams(
            use_tc_tiling_on_sc=True, needs_layout_passes=False
        ),
    )
    def k(idxT_hbm, lt_hbm, out_hbm, idx_v, keyA, keyB, gA, gB, oA, oB,
          sgA, sgB, soA, soB):
        wid = lax.axis_index("s") * info.num_cores + lax.axis_index("c")
        lidx = [_iota() + 16 * t for t in range(8)]

        def addr(n):
            u = wid * per_w + jnp.minimum(n, per_w - 1)
            return u // jblocks, u % jblocks

        def load_keys(n, key_v):
            s, jp = addr(n)
            pltpu.sync_copy(idxT_hbm.at[s, pl.ds(jp * 128, 128)], idx_v)
            pbs = []
            for t in range(8):
                iv = idx_v[pl.ds(16 * t, 16)]
                key_v[pl.ds(16 * t, 16)] = lax.shift_right_logical(iv, 1)
                pbs.append(lax.shift_left(jnp.bitwise_and(iv, 1), 6))
            return tuple(pbs)

        def start_g(key_v, g, sem):
            pltpu.async_copy(lt_hbm.at[key_v], g, sem)

        def wait_g(g, sem):
            pltpu.make_async_copy(lt_hbm.at[pl.ds(0, 128)], g, sem).wait()

        def start_out(n, o, sem):
            s, jp = addr(n)
            pltpu.async_copy(
                o, out_hbm.at[s, pl.ds(0, D), pl.ds(jp * 128, 128)], sem
            )

        def drain_out(o, sem):
            pltpu.make_async_copy(
                o, out_hbm.at[0, pl.ds(0, D), pl.ds(0, 128)], sem
            ).wait()

        def extract(g_v, o_v, pbs):
            @plsc.parallel_loop(0, D, unroll=8)
            def row(c):
                cs = _splat(c)
                for t in range(8):
                    vals = plsc.load_gather(g_v, [lidx[t], pbs[t] + cs])
                    o_v[c, pl.ds(16 * t, 16)] = vals

        pbsA = load_keys(0, keyA)
        start_g(keyA, gA, sgA)
        pbsB = load_keys(1, keyB)
        start_g(keyB, gB, sgB)

        wait_g(gA, sgA)
        extract(gA, oA, pbsA)
        start_out(0, oA, soA)
        pbsA = load_keys(2, keyA)
        start_g(keyA, gA, sgA)

        wait_g(gB, sgB)
        extract(gB, oB, pbsB)
        start_out(1, oB, soB)
        pbsB = load_keys(3, keyB)
        start_g(keyB, gB, sgB)

        def body(i, carry):
            pbsA, pbsB = carry
            n0 = 2 * i
            wait_g(gA, sgA)
            drain_out(oA, soA)
            extract(gA, oA, pbsA)
            start_out(n0, oA, soA)
            pbsA = load_keys(n0 + 2, keyA)
            start_g(keyA, gA, sgA)
            wait_g(gB, sgB)
            drain_out(oB, soB)
            extract(gB, oB, pbsB)
            start_out(n0 + 1, oB, soB)
            pbsB = load_keys(n0 + 3, keyB)
            start_g(keyB, gB, sgB)
            return (pbsA, pbsB)

        lax.fori_loop(1, per_w // 2, body, (pbsA, pbsB))
        wait_g(gA, sgA)
        wait_g(gB, sgB)
        drain_out(oA, soA)
        drain_out(oB, soB)

    return k


def kernel(src_word, table):
    BATCH, SEQ = src_word.shape
    V, D = table.shape
    idxT = src_word.T
    tableT = table.T
    nb_full = V // 128
    tailLT = table[nb_full * 128:].reshape((V - nb_full * 128) // 2, 2 * D)
    lt = _make_detile(V, D)(tableT, tailLT)
    outT = _make_gather(BATCH, SEQ, D, V)(idxT, lt)
    return jnp.transpose(outT, (2, 0, 1))


# speedup vs baseline: 1.6723x; 1.6723x over previous
import functools

import jax
import jax.numpy as jnp
from jax import lax
from jax.experimental import pallas as pl
from jax.experimental.pallas import tpu as pltpu
from jax.experimental.pallas import tpu_sc as plsc

_CHUNK = 640


@functools.cache
def _make_gather(B, D, chunk):
    info = plsc.get_sparse_core_info()
    num_workers = info.num_cores * info.num_subcores
    b_per_w = B // num_workers
    n_chunks = b_per_w // chunk
    mesh = plsc.VectorSubcoreMesh(core_axis_name="c", subcore_axis_name="s")

    @functools.partial(
        pl.kernel,
        mesh=mesh,
        out_type=jax.ShapeDtypeStruct((B, 2 * D), jnp.float32),
        scratch_types=[
            pltpu.VMEM((n_chunks, chunk), jnp.int32),
            pltpu.VMEM((chunk, D), jnp.float32),
            pltpu.VMEM((chunk, D), jnp.float32),
            pltpu.SemaphoreType.DMA,
            pltpu.SemaphoreType.DMA,
            pltpu.SemaphoreType.DMA,
            pltpu.SemaphoreType.DMA,
        ],
        compiler_params=pltpu.CompilerParams(use_tc_tiling_on_sc=False),
    )
    def k(idx_hbm, table_hbm, out_hbm, idx_v, rows0, rows1, sg0, sg1, so0, so1):
        wid = lax.axis_index("s") * info.num_cores + lax.axis_index("c")
        row0 = wid * n_chunks
        pltpu.sync_copy(idx_hbm.at[pl.ds(row0, n_chunks)], idx_v)

        def gather(g, rows, sem):
            return pltpu.async_copy(table_hbm.at[idx_v.at[g]], rows, sem)

        def out_at(g):
            return out_hbm.at[pl.ds((row0 + g) * chunk, chunk), pl.ds(0, D)]

        gather(0, rows0, sg0).wait()
        pltpu.async_copy(rows0, out_at(0), so0)
        gather(1, rows1, sg1).wait()
        pltpu.async_copy(rows1, out_at(1), so1)

        def body(i, carry):
            g = i * 2
            pltpu.make_async_copy(rows0, out_at(g - 2), so0).wait()
            gather(g, rows0, sg0).wait()
            pltpu.async_copy(rows0, out_at(g), so0)
            pltpu.make_async_copy(rows1, out_at(g - 1), so1).wait()
            gather(g + 1, rows1, sg1).wait()
            pltpu.async_copy(rows1, out_at(g + 1), so1)
            return carry

        lax.fori_loop(1, n_chunks // 2, body, 0)
        pltpu.make_async_copy(rows0, out_at(n_chunks - 2), so0).wait()
        pltpu.make_async_copy(rows1, out_at(n_chunks - 1), so1).wait()

    return k


def kernel(src_word, table):
    B = src_word.shape[0] * src_word.shape[1]
    D = table.shape[1]
    idx = src_word.reshape(B // _CHUNK, _CHUNK)
    outP = _make_gather(B, D, _CHUNK)(idx, table)
    outP = outP.reshape(src_word.shape + (2 * D,))
    return outP[:, :, :D]
